# baseline (speedup 1.0000x reference)
import functools

import jax
import jax.numpy as jnp
from jax import lax
from jax.experimental import pallas as pl
from jax.experimental.pallas import tpu as pltpu
from jax.experimental.pallas import tpu_sc as plsc

T = 2048
H = 1024
E = 8
FFN = 512
SFFN = 2048
NGROUP = 4
BC = 512
NJS = SFFN // BC
R = 2 * T
BT = 256
NT = R // BT
S = NT + E
NEG = -1e30

NC = 2
NS = 16
NW = NC * NS
TPW = T // NW
CH = 32
VPR = H // 16


def _router_kernel(scores_ref, bias_ref, p1_ref, p2_ref, w1_ref, w2_ref,
                   counts_ref):
    scores = scores_ref[...]
    sfc = scores + bias_ref[...]

    eidx = lax.broadcasted_iota(jnp.int32, (T, E), 1)
    gidx8 = eidx // 2
    gsum_full = jnp.zeros((T, E), jnp.float32)
    for g in range(NGROUP):
        in_g = gidx8 == g
        gsum_g = jnp.sum(jnp.where(in_g, sfc, 0.0), axis=-1, keepdims=True)
        gsum_full = jnp.where(in_g, gsum_g, gsum_full)

    m1 = jnp.max(gsum_full, axis=-1, keepdims=True)
    i1 = jnp.min(jnp.where(gsum_full == m1, gidx8, NGROUP),
                 axis=-1, keepdims=True)
    gs2 = jnp.where(gidx8 == i1, NEG, gsum_full)
    m2 = jnp.max(gs2, axis=-1, keepdims=True)
    i2 = jnp.min(jnp.where(gs2 == m2, gidx8, NGROUP),
                 axis=-1, keepdims=True)
    gsel = (gidx8 == i1) | (gidx8 == i2)
    masked = jnp.where(gsel, sfc, NEG)

    e_m1 = jnp.max(masked, axis=-1, keepdims=True)
    e1 = jnp.min(jnp.where(masked == e_m1, eidx, E), axis=-1, keepdims=True)
    masked2 = jnp.where(eidx == e1, NEG, masked)
    e_m2 = jnp.max(masked2, axis=-1, keepdims=True)
    e2 = jnp.min(jnp.where(masked2 == e_m2, eidx, E), axis=-1, keepdims=True)

    sel1 = (eidx == e1)
    sel2 = (eidx == e2)
    w1 = jnp.sum(jnp.where(sel1, scores, 0.0), axis=-1, keepdims=True)
    w2 = jnp.sum(jnp.where(sel2, scores, 0.0), axis=-1, keepdims=True)
    denom = w1 + w2 + 1e-20
    w1_ref[...] = jnp.broadcast_to(w1 / denom, (T, 128))
    w2_ref[...] = jnp.broadcast_to(w2 / denom, (T, 128))

    mask = (sel1 | sel2).astype(jnp.float32)
    r_t = lax.broadcasted_iota(jnp.int32, (T, T), 0)
    c_t = lax.broadcasted_iota(jnp.int32, (T, T), 1)
    ltri = (c_t < r_t).astype(jnp.float32)
    rank = lax.dot_general(ltri, mask, (((1,), (0,)), ((), ())),
                           preferred_element_type=jnp.float32)
    counts = jnp.sum(mask, axis=0, keepdims=True)
    lidx = lax.broadcasted_iota(jnp.int32, (1, E), 1)
    offs = jnp.zeros((1, E), jnp.float32)
    for e in range(1, E):
        off_e = jnp.sum(jnp.where(lidx < e, counts, 0.0),
                        axis=-1, keepdims=True)
        offs = jnp.where(lidx == e, off_e, offs)
    pos = offs + rank
    p1_ref[...] = jnp.sum(jnp.where(sel1, pos, 0.0), axis=-1,
                          keepdims=True).astype(jnp.int32)
    p2_ref[...] = jnp.sum(jnp.where(sel2, pos, 0.0), axis=-1,
                          keepdims=True).astype(jnp.int32)
    counts_ref[...] = counts.astype(jnp.int32)


def _gmm_kernel(tid_ref, eid_ref, lo_ref, hi_ref, first_ref,
                xs_ref, ws_ref, wg_ref, wu_ref, wd_ref, out_ref):
    s = pl.program_id(0)
    xs = xs_ref[...].astype(jnp.bfloat16)
    wg = wg_ref[0].astype(jnp.bfloat16)
    wu = wu_ref[0].astype(jnp.bfloat16)
    wd = wd_ref[0].astype(jnp.bfloat16)
    g = lax.dot_general(xs, wg, (((1,), (1,)), ((), ())),
                        preferred_element_type=jnp.float32)
    u = lax.dot_general(xs, wu, (((1,), (1,)), ((), ())),
                        preferred_element_type=jnp.float32)
    wv = jnp.max(ws_ref[...], axis=-1, keepdims=True)
    ridx = lax.broadcasted_iota(jnp.int32, (BT, FFN), 0)
    live = (ridx >= lo_ref[s]) & (ridx < hi_ref[s])
    a = jnp.where(live, g * jax.nn.sigmoid(g) * u * wv,
                  0.0).astype(jnp.bfloat16)
    partial = lax.dot_general(a, wd, (((1,), (1,)), ((), ())),
                              preferred_element_type=jnp.float32)

    @pl.when(first_ref[s] != 0)
    def _():
        out_ref[...] = partial

    @pl.when(first_ref[s] == 0)
    def _():
        out_ref[...] += partial


def _shared_kernel(x_ref, wg_ref, wu_ref, wd_ref, out_ref):
    x = x_ref[...]
    wg = wg_ref[...].astype(jnp.bfloat16)
    wu = wu_ref[...].astype(jnp.bfloat16)
    wd = wd_ref[...].astype(jnp.bfloat16)
    g = lax.dot_general(x, wg, (((1,), (1,)), ((), ())),
                        preferred_element_type=jnp.float32)
    u = lax.dot_general(x, wu, (((1,), (1,)), ((), ())),
                        preferred_element_type=jnp.float32)
    a = (g * jax.nn.sigmoid(g) * u).astype(jnp.bfloat16)
    partial = lax.dot_general(a, wd, (((1,), (1,)), ((), ())),
                              preferred_element_type=jnp.float32)

    @pl.when(pl.program_id(0) == 0)
    def _():
        out_ref[...] = partial

    @pl.when(pl.program_id(0) > 0)
    def _():
        out_ref[...] += partial


def _dispatch_body(x_hbm, p1_hbm, p2_hbm, w1_hbm, w2_hbm, xs_hbm, ws_hbm,
                   idx1_v, idx2_v, rows_v, wrows_v, sem):
    wid = lax.axis_index("s") * NC + lax.axis_index("c")
    base = wid * TPW
    pltpu.sync_copy(p1_hbm.at[pl.ds(base, TPW)], idx1_v)
    pltpu.sync_copy(p2_hbm.at[pl.ds(base, TPW)], idx2_v)
    pltpu.sync_copy(x_hbm.at[pl.ds(base, TPW)], rows_v)
    pltpu.async_copy(rows_v, xs_hbm.at[idx1_v], sem).wait()
    pltpu.async_copy(rows_v, xs_hbm.at[idx2_v], sem).wait()
    pltpu.sync_copy(w1_hbm.at[pl.ds(base, TPW)], wrows_v)
    pltpu.async_copy(wrows_v, ws_hbm.at[idx1_v], sem).wait()
    pltpu.sync_copy(w2_hbm.at[pl.ds(base, TPW)], wrows_v)
    pltpu.async_copy(wrows_v, ws_hbm.at[idx2_v], sem).wait()


def _combine_body(douts_hbm, shared_hbm, p1_hbm, p2_hbm,
                  out_hbm, idx1_v, idx2_v, r1_v, r2_v, sh_v, sem):
    wid = lax.axis_index("s") * NC + lax.axis_index("c")
    for c in range(TPW // CH):
        cbase = wid * TPW + c * CH
        pltpu.sync_copy(p1_hbm.at[pl.ds(cbase, CH)], idx1_v)
        pltpu.sync_copy(p2_hbm.at[pl.ds(cbase, CH)], idx2_v)
        pltpu.async_copy(douts_hbm.at[idx1_v], r1_v, sem).wait()
        pltpu.async_copy(douts_hbm.at[idx2_v], r2_v, sem).wait()
        pltpu.sync_copy(shared_hbm.at[pl.ds(cbase, CH)], sh_v)

        def tok(i, _):
            for j in range(VPR):
                sl = pl.ds(j * 16, 16)
                sh_v[i, sl] = sh_v[i, sl] + r1_v[i, sl] + r2_v[i, sl]
            return 0

        lax.fori_loop(0, CH, tok, 0)
        pltpu.sync_copy(sh_v, out_hbm.at[pl.ds(cbase, CH)])


def _step_table(counts):
    b = jnp.concatenate([jnp.zeros((1,), jnp.int32),
                         jnp.cumsum(counts)]).astype(jnp.int32)
    ii = jnp.arange(NT, dtype=jnp.int32)[:, None]
    ee = jnp.arange(E, dtype=jnp.int32)[None, :]
    lo = jnp.maximum(b[ee], BT * ii)
    hi = jnp.minimum(b[ee + 1], BT * ii + BT)
    act = lo < hi
    posn = jnp.cumsum(act.reshape(-1)).astype(jnp.int32) - 1
    wpos = jnp.where(act.reshape(-1), posn, S)
    first = act & (jnp.cumsum(act, axis=1) == 1)

    def scat(vals, fill):
        return jnp.full((S,), fill, jnp.int32).at[wpos].set(
            vals.reshape(-1).astype(jnp.int32), mode="drop")

    t_idx = scat(jnp.broadcast_to(ii, (NT, E)), NT - 1)
    e_idx = scat(jnp.broadcast_to(ee, (NT, E)), E - 1)
    lo_r = scat(lo - BT * ii, 0)
    hi_r = scat(hi - BT * ii, 0)
    first_i = scat(first, 0)
    return t_idx, e_idx, lo_r, hi_r, first_i


def kernel(hidden_states, gate_weight, e_score_correction_bias, expert_gate,
           expert_up, expert_down, shared_gate, shared_up, shared_down):
    x = hidden_states.reshape(T, H)
    scores = jax.nn.sigmoid(x @ gate_weight.astype(jnp.float32).T)

    p1, p2, w1, w2, counts = pl.pallas_call(
        _router_kernel,
        out_shape=[
            jax.ShapeDtypeStruct((T, 1), jnp.int32),
            jax.ShapeDtypeStruct((T, 1), jnp.int32),
            jax.ShapeDtypeStruct((T, 128), jnp.float32),
            jax.ShapeDtypeStruct((T, 128), jnp.float32),
            jax.ShapeDtypeStruct((1, E), jnp.int32),
        ],
    )(scores, e_score_correction_bias.reshape(1, E))
    p1 = p1.reshape(T)
    p2 = p2.reshape(T)

    t_idx, e_idx, lo_r, hi_r, first_i = _step_table(counts.reshape(E))

    mesh = plsc.VectorSubcoreMesh(core_axis_name="c", subcore_axis_name="s")
    xs, ws = pl.kernel(
        _dispatch_body,
        out_type=[
            jax.ShapeDtypeStruct((R, H), jnp.float32),
            jax.ShapeDtypeStruct((R, 128), jnp.float32),
        ],
        mesh=mesh,
        scratch_types=[
            pltpu.VMEM((TPW,), jnp.int32),
            pltpu.VMEM((TPW,), jnp.int32),
            pltpu.VMEM((TPW, H), jnp.float32),
            pltpu.VMEM((TPW, 128), jnp.float32),
            pltpu.SemaphoreType.DMA,
        ],
    )(x, p1, p2, w1, w2)

    douts = pl.pallas_call(
        _gmm_kernel,
        grid_spec=pltpu.PrefetchScalarGridSpec(
            num_scalar_prefetch=5,
            grid=(S,),
            in_specs=[
                pl.BlockSpec((BT, H), lambda s, t, e, lo, hi, f: (t[s], 0)),
                pl.BlockSpec((BT, 128), lambda s, t, e, lo, hi, f: (t[s], 0)),
                pl.BlockSpec((1, FFN, H),
                             lambda s, t, e, lo, hi, f: (e[s], 0, 0)),
                pl.BlockSpec((1, FFN, H),
                             lambda s, t, e, lo, hi, f: (e[s], 0, 0)),
                pl.BlockSpec((1, H, FFN),
                             lambda s, t, e, lo, hi, f: (e[s], 0, 0)),
            ],
            out_specs=pl.BlockSpec((BT, H),
                                   lambda s, t, e, lo, hi, f: (t[s], 0)),
        ),
        out_shape=jax.ShapeDtypeStruct((R, H), jnp.float32),
    )(t_idx, e_idx, lo_r, hi_r, first_i, xs, ws, expert_gate, expert_up,
      expert_down)

    xb = x.astype(jnp.bfloat16)
    shared = pl.pallas_call(
        _shared_kernel,
        grid=(NJS,),
        in_specs=[
            pl.BlockSpec((T, H), lambda j: (0, 0)),
            pl.BlockSpec((BC, H), lambda j: (j, 0)),
            pl.BlockSpec((BC, H), lambda j: (j, 0)),
            pl.BlockSpec((H, BC), lambda j: (0, j)),
        ],
        out_specs=pl.BlockSpec((T, H), lambda j: (0, 0)),
        out_shape=jax.ShapeDtypeStruct((T, H), jnp.float32),
    )(xb, shared_gate, shared_up, shared_down)

    out = pl.kernel(
        _combine_body,
        out_type=jax.ShapeDtypeStruct((T, H), jnp.float32),
        mesh=mesh,
        scratch_types=[
            pltpu.VMEM((CH,), jnp.int32),
            pltpu.VMEM((CH,), jnp.int32),
            pltpu.VMEM((CH, H), jnp.float32),
            pltpu.VMEM((CH, H), jnp.float32),
            pltpu.VMEM((CH, H), jnp.float32),
            pltpu.SemaphoreType.DMA,
        ],
    )(douts, shared, p1, p2)

    return out.reshape(1, T, H)


# speedup vs baseline: 1.0567x; 1.0567x over previous
import functools

import jax
import jax.numpy as jnp
from jax import lax
from jax.experimental import pallas as pl
from jax.experimental.pallas import tpu as pltpu
from jax.experimental.pallas import tpu_sc as plsc

T = 2048
H = 1024
E = 8
FFN = 512
SFFN = 2048
NGROUP = 4
BC = 512
NJS = SFFN // BC
R = 2 * T
BT = 256
NT = R // BT
S = NT + E
NEG = -1e30

NC = 2
NS = 16
NW = NC * NS
TPW = T // NW
CH = 32
VPR = H // 16


def _router_kernel(scores_ref, bias_ref, p1_ref, p2_ref, w1_ref, w2_ref,
                   counts_ref):
    scores = scores_ref[...]
    sfc = scores + bias_ref[...]

    eidx = lax.broadcasted_iota(jnp.int32, (T, E), 1)
    gidx8 = eidx // 2
    gsum_full = jnp.zeros((T, E), jnp.float32)
    for g in range(NGROUP):
        in_g = gidx8 == g
        gsum_g = jnp.sum(jnp.where(in_g, sfc, 0.0), axis=-1, keepdims=True)
        gsum_full = jnp.where(in_g, gsum_g, gsum_full)

    m1 = jnp.max(gsum_full, axis=-1, keepdims=True)
    i1 = jnp.min(jnp.where(gsum_full == m1, gidx8, NGROUP),
                 axis=-1, keepdims=True)
    gs2 = jnp.where(gidx8 == i1, NEG, gsum_full)
    m2 = jnp.max(gs2, axis=-1, keepdims=True)
    i2 = jnp.min(jnp.where(gs2 == m2, gidx8, NGROUP),
                 axis=-1, keepdims=True)
    gsel = (gidx8 == i1) | (gidx8 == i2)
    masked = jnp.where(gsel, sfc, NEG)

    e_m1 = jnp.max(masked, axis=-1, keepdims=True)
    e1 = jnp.min(jnp.where(masked == e_m1, eidx, E), axis=-1, keepdims=True)
    masked2 = jnp.where(eidx == e1, NEG, masked)
    e_m2 = jnp.max(masked2, axis=-1, keepdims=True)
    e2 = jnp.min(jnp.where(masked2 == e_m2, eidx, E), axis=-1, keepdims=True)

    sel1 = (eidx == e1)
    sel2 = (eidx == e2)
    w1 = jnp.sum(jnp.where(sel1, scores, 0.0), axis=-1, keepdims=True)
    w2 = jnp.sum(jnp.where(sel2, scores, 0.0), axis=-1, keepdims=True)
    denom = w1 + w2 + 1e-20
    w1_ref[...] = jnp.broadcast_to(w1 / denom, (T, 128))
    w2_ref[...] = jnp.broadcast_to(w2 / denom, (T, 128))

    mask = (sel1 | sel2).astype(jnp.float32)
    r_t = lax.broadcasted_iota(jnp.int32, (T, T), 0)
    c_t = lax.broadcasted_iota(jnp.int32, (T, T), 1)
    ltri = (c_t < r_t).astype(jnp.float32)
    rank = lax.dot_general(ltri, mask, (((1,), (0,)), ((), ())),
                           preferred_element_type=jnp.float32)
    counts = jnp.sum(mask, axis=0, keepdims=True)
    lidx = lax.broadcasted_iota(jnp.int32, (1, E), 1)
    offs = jnp.zeros((1, E), jnp.float32)
    for e in range(1, E):
        off_e = jnp.sum(jnp.where(lidx < e, counts, 0.0),
                        axis=-1, keepdims=True)
        offs = jnp.where(lidx == e, off_e, offs)
    pos = offs + rank
    p1_ref[...] = jnp.sum(jnp.where(sel1, pos, 0.0), axis=-1,
                          keepdims=True).astype(jnp.int32)
    p2_ref[...] = jnp.sum(jnp.where(sel2, pos, 0.0), axis=-1,
                          keepdims=True).astype(jnp.int32)
    counts_ref[...] = counts.astype(jnp.int32)


def _gmm_kernel(tid_ref, eid_ref, lo_ref, hi_ref, first_ref,
                xs_ref, ws_ref, wg_ref, wu_ref, wd_ref, out_ref):
    s = pl.program_id(0)
    xs = xs_ref[...].astype(jnp.bfloat16)
    wg = wg_ref[0].astype(jnp.bfloat16)
    wu = wu_ref[0].astype(jnp.bfloat16)
    wd = wd_ref[0].astype(jnp.bfloat16)
    g = lax.dot_general(xs, wg, (((1,), (1,)), ((), ())),
                        preferred_element_type=jnp.float32)
    u = lax.dot_general(xs, wu, (((1,), (1,)), ((), ())),
                        preferred_element_type=jnp.float32)
    wv = jnp.max(ws_ref[...], axis=-1, keepdims=True)
    ridx = lax.broadcasted_iota(jnp.int32, (BT, FFN), 0)
    live = (ridx >= lo_ref[s]) & (ridx < hi_ref[s])
    a = jnp.where(live, g * jax.nn.sigmoid(g) * u * wv,
                  0.0).astype(jnp.bfloat16)
    partial = lax.dot_general(a, wd, (((1,), (1,)), ((), ())),
                              preferred_element_type=jnp.float32)

    @pl.when(first_ref[s] != 0)
    def _():
        out_ref[...] = partial

    @pl.when(first_ref[s] == 0)
    def _():
        out_ref[...] += partial


def _shared_kernel(x_ref, wg_ref, wu_ref, wd_ref, out_ref):
    x = x_ref[...]
    wg = wg_ref[...].astype(jnp.bfloat16)
    wu = wu_ref[...].astype(jnp.bfloat16)
    wd = wd_ref[...].astype(jnp.bfloat16)
    g = lax.dot_general(x, wg, (((1,), (1,)), ((), ())),
                        preferred_element_type=jnp.float32)
    u = lax.dot_general(x, wu, (((1,), (1,)), ((), ())),
                        preferred_element_type=jnp.float32)
    a = (g * jax.nn.sigmoid(g) * u).astype(jnp.bfloat16)
    partial = lax.dot_general(a, wd, (((1,), (1,)), ((), ())),
                              preferred_element_type=jnp.float32)

    @pl.when(pl.program_id(0) == 0)
    def _():
        out_ref[...] = partial

    @pl.when(pl.program_id(0) > 0)
    def _():
        out_ref[...] += partial


def _dispatch_body(x_hbm, p1_hbm, p2_hbm, w1_hbm, w2_hbm, xs_hbm, ws_hbm,
                   idx1_v, idx2_v, rows_v, wrows_v, sem):
    wid = lax.axis_index("s") * NC + lax.axis_index("c")
    base = wid * TPW
    pltpu.sync_copy(p1_hbm.at[pl.ds(base, TPW)], idx1_v)
    pltpu.sync_copy(p2_hbm.at[pl.ds(base, TPW)], idx2_v)
    pltpu.sync_copy(x_hbm.at[pl.ds(base, TPW)], rows_v)
    pltpu.async_copy(rows_v, xs_hbm.at[idx1_v], sem).wait()
    pltpu.async_copy(rows_v, xs_hbm.at[idx2_v], sem).wait()
    pltpu.sync_copy(w1_hbm.at[pl.ds(base, TPW)], wrows_v)
    pltpu.async_copy(wrows_v, ws_hbm.at[idx1_v], sem).wait()
    pltpu.sync_copy(w2_hbm.at[pl.ds(base, TPW)], wrows_v)
    pltpu.async_copy(wrows_v, ws_hbm.at[idx2_v], sem).wait()


def _combine_body(douts_hbm, p1_hbm, p2_hbm,
                  rp1_hbm, rp2_hbm, idx1_v, idx2_v, r1_v, r2_v, sem):
    wid = lax.axis_index("s") * NC + lax.axis_index("c")
    for c in range(TPW // CH):
        cbase = wid * TPW + c * CH
        pltpu.sync_copy(p1_hbm.at[pl.ds(cbase, CH)], idx1_v)
        pltpu.sync_copy(p2_hbm.at[pl.ds(cbase, CH)], idx2_v)
        c1 = pltpu.async_copy(douts_hbm.at[idx1_v], r1_v, sem)
        c2 = pltpu.async_copy(douts_hbm.at[idx2_v], r2_v, sem)
        c1.wait()
        pltpu.sync_copy(r1_v, rp1_hbm.at[pl.ds(cbase, CH)])
        c2.wait()
        pltpu.sync_copy(r2_v, rp2_hbm.at[pl.ds(cbase, CH)])


def _add_kernel(shared_ref, r1_ref, r2_ref, out_ref):
    out_ref[...] = shared_ref[...] + r1_ref[...] + r2_ref[...]


def _step_table(counts):
    b = jnp.concatenate([jnp.zeros((1,), jnp.int32),
                         jnp.cumsum(counts)]).astype(jnp.int32)
    ii = jnp.arange(NT, dtype=jnp.int32)[:, None]
    ee = jnp.arange(E, dtype=jnp.int32)[None, :]
    lo = jnp.maximum(b[ee], BT * ii)
    hi = jnp.minimum(b[ee + 1], BT * ii + BT)
    act = lo < hi
    posn = jnp.cumsum(act.reshape(-1)).astype(jnp.int32) - 1
    wpos = jnp.where(act.reshape(-1), posn, S)
    first = act & (jnp.cumsum(act, axis=1) == 1)

    def scat(vals, fill):
        return jnp.full((S,), fill, jnp.int32).at[wpos].set(
            vals.reshape(-1).astype(jnp.int32), mode="drop")

    t_idx = scat(jnp.broadcast_to(ii, (NT, E)), NT - 1)
    e_idx = scat(jnp.broadcast_to(ee, (NT, E)), E - 1)
    lo_r = scat(lo - BT * ii, 0)
    hi_r = scat(hi - BT * ii, 0)
    first_i = scat(first, 0)
    return t_idx, e_idx, lo_r, hi_r, first_i


def kernel(hidden_states, gate_weight, e_score_correction_bias, expert_gate,
           expert_up, expert_down, shared_gate, shared_up, shared_down):
    x = hidden_states.reshape(T, H)
    scores = jax.nn.sigmoid(x @ gate_weight.astype(jnp.float32).T)

    p1, p2, w1, w2, counts = pl.pallas_call(
        _router_kernel,
        out_shape=[
            jax.ShapeDtypeStruct((T, 1), jnp.int32),
            jax.ShapeDtypeStruct((T, 1), jnp.int32),
            jax.ShapeDtypeStruct((T, 128), jnp.float32),
            jax.ShapeDtypeStruct((T, 128), jnp.float32),
            jax.ShapeDtypeStruct((1, E), jnp.int32),
        ],
    )(scores, e_score_correction_bias.reshape(1, E))
    p1 = p1.reshape(T)
    p2 = p2.reshape(T)

    t_idx, e_idx, lo_r, hi_r, first_i = _step_table(counts.reshape(E))

    mesh = plsc.VectorSubcoreMesh(core_axis_name="c", subcore_axis_name="s")
    xs, ws = pl.kernel(
        _dispatch_body,
        out_type=[
            jax.ShapeDtypeStruct((R, H), jnp.float32),
            jax.ShapeDtypeStruct((R, 128), jnp.float32),
        ],
        mesh=mesh,
        scratch_types=[
            pltpu.VMEM((TPW,), jnp.int32),
            pltpu.VMEM((TPW,), jnp.int32),
            pltpu.VMEM((TPW, H), jnp.float32),
            pltpu.VMEM((TPW, 128), jnp.float32),
            pltpu.SemaphoreType.DMA,
        ],
    )(x, p1, p2, w1, w2)

    douts = pl.pallas_call(
        _gmm_kernel,
        grid_spec=pltpu.PrefetchScalarGridSpec(
            num_scalar_prefetch=5,
            grid=(S,),
            in_specs=[
                pl.BlockSpec((BT, H), lambda s, t, e, lo, hi, f: (t[s], 0)),
                pl.BlockSpec((BT, 128), lambda s, t, e, lo, hi, f: (t[s], 0)),
                pl.BlockSpec((1, FFN, H),
                             lambda s, t, e, lo, hi, f: (e[s], 0, 0)),
                pl.BlockSpec((1, FFN, H),
                             lambda s, t, e, lo, hi, f: (e[s], 0, 0)),
                pl.BlockSpec((1, H, FFN),
                             lambda s, t, e, lo, hi, f: (e[s], 0, 0)),
            ],
            out_specs=pl.BlockSpec((BT, H),
                                   lambda s, t, e, lo, hi, f: (t[s], 0)),
        ),
        out_shape=jax.ShapeDtypeStruct((R, H), jnp.float32),
    )(t_idx, e_idx, lo_r, hi_r, first_i, xs, ws, expert_gate, expert_up,
      expert_down)

    xb = x.astype(jnp.bfloat16)
    shared = pl.pallas_call(
        _shared_kernel,
        grid=(NJS,),
        in_specs=[
            pl.BlockSpec((T, H), lambda j: (0, 0)),
            pl.BlockSpec((BC, H), lambda j: (j, 0)),
            pl.BlockSpec((BC, H), lambda j: (j, 0)),
            pl.BlockSpec((H, BC), lambda j: (0, j)),
        ],
        out_specs=pl.BlockSpec((T, H), lambda j: (0, 0)),
        out_shape=jax.ShapeDtypeStruct((T, H), jnp.float32),
    )(xb, shared_gate, shared_up, shared_down)

    rp1, rp2 = pl.kernel(
        _combine_body,
        out_type=[
            jax.ShapeDtypeStruct((T, H), jnp.float32),
            jax.ShapeDtypeStruct((T, H), jnp.float32),
        ],
        mesh=mesh,
        scratch_types=[
            pltpu.VMEM((CH,), jnp.int32),
            pltpu.VMEM((CH,), jnp.int32),
            pltpu.VMEM((CH, H), jnp.float32),
            pltpu.VMEM((CH, H), jnp.float32),
            pltpu.SemaphoreType.DMA,
        ],
    )(douts, p1, p2)

    out = pl.pallas_call(
        _add_kernel,
        grid=(4,),
        in_specs=[
            pl.BlockSpec((T // 4, H), lambda j: (j, 0)),
            pl.BlockSpec((T // 4, H), lambda j: (j, 0)),
            pl.BlockSpec((T // 4, H), lambda j: (j, 0)),
        ],
        out_specs=pl.BlockSpec((T // 4, H), lambda j: (j, 0)),
        out_shape=jax.ShapeDtypeStruct((T, H), jnp.float32),
    )(shared, rp1, rp2)

    return out.reshape(1, T, H)


# speedup vs baseline: 1.0768x; 1.0190x over previous
import functools

import jax
import jax.numpy as jnp
from jax import lax
from jax.experimental import pallas as pl
from jax.experimental.pallas import tpu as pltpu
from jax.experimental.pallas import tpu_sc as plsc

T = 2048
H = 1024
E = 8
FFN = 512
SFFN = 2048
NGROUP = 4
BC = 512
NJS = SFFN // BC
R = 2 * T
BT = 256
NT = R // BT
S = NT + E
NEG = -1e30

NC = 2
NS = 16
NW = NC * NS
TPW = T // NW
CH = 32
VPR = H // 16


def _router_kernel(scores_ref, bias_ref, p1_ref, p2_ref, w1_ref, w2_ref,
                   counts_ref):
    scores = scores_ref[...]
    sfc = scores + bias_ref[...]

    eidx = lax.broadcasted_iota(jnp.int32, (T, E), 1)
    gidx8 = eidx // 2
    gsum_full = jnp.zeros((T, E), jnp.float32)
    for g in range(NGROUP):
        in_g = gidx8 == g
        gsum_g = jnp.sum(jnp.where(in_g, sfc, 0.0), axis=-1, keepdims=True)
        gsum_full = jnp.where(in_g, gsum_g, gsum_full)

    m1 = jnp.max(gsum_full, axis=-1, keepdims=True)
    i1 = jnp.min(jnp.where(gsum_full == m1, gidx8, NGROUP),
                 axis=-1, keepdims=True)
    gs2 = jnp.where(gidx8 == i1, NEG, gsum_full)
    m2 = jnp.max(gs2, axis=-1, keepdims=True)
    i2 = jnp.min(jnp.where(gs2 == m2, gidx8, NGROUP),
                 axis=-1, keepdims=True)
    gsel = (gidx8 == i1) | (gidx8 == i2)
    masked = jnp.where(gsel, sfc, NEG)

    e_m1 = jnp.max(masked, axis=-1, keepdims=True)
    e1 = jnp.min(jnp.where(masked == e_m1, eidx, E), axis=-1, keepdims=True)
    masked2 = jnp.where(eidx == e1, NEG, masked)
    e_m2 = jnp.max(masked2, axis=-1, keepdims=True)
    e2 = jnp.min(jnp.where(masked2 == e_m2, eidx, E), axis=-1, keepdims=True)

    sel1 = (eidx == e1)
    sel2 = (eidx == e2)
    w1 = jnp.sum(jnp.where(sel1, scores, 0.0), axis=-1, keepdims=True)
    w2 = jnp.sum(jnp.where(sel2, scores, 0.0), axis=-1, keepdims=True)
    denom = w1 + w2 + 1e-20
    w1_ref[...] = jnp.broadcast_to(w1 / denom, (T, 128))
    w2_ref[...] = jnp.broadcast_to(w2 / denom, (T, 128))

    mask = (sel1 | sel2).astype(jnp.float32)
    r_t = lax.broadcasted_iota(jnp.int32, (T, T), 0)
    c_t = lax.broadcasted_iota(jnp.int32, (T, T), 1)
    ltri = (c_t < r_t).astype(jnp.float32)
    rank = lax.dot_general(ltri, mask, (((1,), (0,)), ((), ())),
                           preferred_element_type=jnp.float32)
    counts = jnp.sum(mask, axis=0, keepdims=True)
    lidx = lax.broadcasted_iota(jnp.int32, (1, E), 1)
    offs = jnp.zeros((1, E), jnp.float32)
    for e in range(1, E):
        off_e = jnp.sum(jnp.where(lidx < e, counts, 0.0),
                        axis=-1, keepdims=True)
        offs = jnp.where(lidx == e, off_e, offs)
    pos = offs + rank
    p1_ref[...] = jnp.sum(jnp.where(sel1, pos, 0.0), axis=-1,
                          keepdims=True).astype(jnp.int32)
    p2_ref[...] = jnp.sum(jnp.where(sel2, pos, 0.0), axis=-1,
                          keepdims=True).astype(jnp.int32)
    counts_ref[...] = counts.astype(jnp.int32)


def _gmm_kernel(tid_ref, eid_ref, lo_ref, hi_ref, first_ref,
                xs_ref, ws_ref, wg_ref, wu_ref, wd_ref, out_ref):
    s = pl.program_id(0)
    xs = xs_ref[...].astype(jnp.bfloat16)
    wg = wg_ref[0].astype(jnp.bfloat16)
    wu = wu_ref[0].astype(jnp.bfloat16)
    wd = wd_ref[0].astype(jnp.bfloat16)
    g = lax.dot_general(xs, wg, (((1,), (1,)), ((), ())),
                        preferred_element_type=jnp.float32)
    u = lax.dot_general(xs, wu, (((1,), (1,)), ((), ())),
                        preferred_element_type=jnp.float32)
    wv = jnp.max(ws_ref[...], axis=-1, keepdims=True)
    ridx = lax.broadcasted_iota(jnp.int32, (BT, FFN), 0)
    live = (ridx >= lo_ref[s]) & (ridx < hi_ref[s])
    a = jnp.where(live, g * jax.nn.sigmoid(g) * u * wv,
                  0.0).astype(jnp.bfloat16)
    partial = lax.dot_general(a, wd, (((1,), (1,)), ((), ())),
                              preferred_element_type=jnp.float32)

    @pl.when(first_ref[s] != 0)
    def _():
        out_ref[...] = partial

    @pl.when(first_ref[s] == 0)
    def _():
        out_ref[...] += partial


def _shared_kernel(x_ref, r1_ref, r2_ref, wg_ref, wu_ref, wd_ref, out_ref):
    x = x_ref[...]
    wg = wg_ref[...].astype(jnp.bfloat16)
    wu = wu_ref[...].astype(jnp.bfloat16)
    wd = wd_ref[...].astype(jnp.bfloat16)
    g = lax.dot_general(x, wg, (((1,), (1,)), ((), ())),
                        preferred_element_type=jnp.float32)
    u = lax.dot_general(x, wu, (((1,), (1,)), ((), ())),
                        preferred_element_type=jnp.float32)
    a = (g * jax.nn.sigmoid(g) * u).astype(jnp.bfloat16)
    partial = lax.dot_general(a, wd, (((1,), (1,)), ((), ())),
                              preferred_element_type=jnp.float32)

    @pl.when(pl.program_id(0) == 0)
    def _():
        out_ref[...] = partial + r1_ref[...] + r2_ref[...]

    @pl.when(pl.program_id(0) > 0)
    def _():
        out_ref[...] += partial


def _dispatch_body(x_hbm, p1_hbm, p2_hbm, w1_hbm, w2_hbm, xs_hbm, ws_hbm,
                   idx1_v, idx2_v, rows_v, wrows_v, sem):
    wid = lax.axis_index("s") * NC + lax.axis_index("c")
    base = wid * TPW
    pltpu.sync_copy(p1_hbm.at[pl.ds(base, TPW)], idx1_v)
    pltpu.sync_copy(p2_hbm.at[pl.ds(base, TPW)], idx2_v)
    pltpu.sync_copy(x_hbm.at[pl.ds(base, TPW)], rows_v)
    pltpu.async_copy(rows_v, xs_hbm.at[idx1_v], sem).wait()
    pltpu.async_copy(rows_v, xs_hbm.at[idx2_v], sem).wait()
    pltpu.sync_copy(w1_hbm.at[pl.ds(base, TPW)], wrows_v)
    pltpu.async_copy(wrows_v, ws_hbm.at[idx1_v], sem).wait()
    pltpu.sync_copy(w2_hbm.at[pl.ds(base, TPW)], wrows_v)
    pltpu.async_copy(wrows_v, ws_hbm.at[idx2_v], sem).wait()


def _combine_body(douts_hbm, p1_hbm, p2_hbm,
                  rp1_hbm, rp2_hbm, idx1_v, idx2_v, r1_v, r2_v, sem):
    wid = lax.axis_index("s") * NC + lax.axis_index("c")
    for c in range(TPW // CH):
        cbase = wid * TPW + c * CH
        pltpu.sync_copy(p1_hbm.at[pl.ds(cbase, CH)], idx1_v)
        pltpu.sync_copy(p2_hbm.at[pl.ds(cbase, CH)], idx2_v)
        c1 = pltpu.async_copy(douts_hbm.at[idx1_v], r1_v, sem)
        c2 = pltpu.async_copy(douts_hbm.at[idx2_v], r2_v, sem)
        c1.wait()
        pltpu.sync_copy(r1_v, rp1_hbm.at[pl.ds(cbase, CH)])
        c2.wait()
        pltpu.sync_copy(r2_v, rp2_hbm.at[pl.ds(cbase, CH)])


def _step_table(counts):
    b = jnp.concatenate([jnp.zeros((1,), jnp.int32),
                         jnp.cumsum(counts)]).astype(jnp.int32)
    ii = jnp.arange(NT, dtype=jnp.int32)[:, None]
    ee = jnp.arange(E, dtype=jnp.int32)[None, :]
    lo = jnp.maximum(b[ee], BT * ii)
    hi = jnp.minimum(b[ee + 1], BT * ii + BT)
    act = lo < hi
    posn = jnp.cumsum(act.reshape(-1)).astype(jnp.int32) - 1
    wpos = jnp.where(act.reshape(-1), posn, S)
    first = act & (jnp.cumsum(act, axis=1) == 1)

    def scat(vals, fill):
        return jnp.full((S,), fill, jnp.int32).at[wpos].set(
            vals.reshape(-1).astype(jnp.int32), mode="drop")

    t_idx = scat(jnp.broadcast_to(ii, (NT, E)), NT - 1)
    e_idx = scat(jnp.broadcast_to(ee, (NT, E)), E - 1)
    lo_r = scat(lo - BT * ii, 0)
    hi_r = scat(hi - BT * ii, 0)
    first_i = scat(first, 0)
    return t_idx, e_idx, lo_r, hi_r, first_i


def kernel(hidden_states, gate_weight, e_score_correction_bias, expert_gate,
           expert_up, expert_down, shared_gate, shared_up, shared_down):
    x = hidden_states.reshape(T, H)
    scores = jax.nn.sigmoid(x @ gate_weight.astype(jnp.float32).T)

    p1, p2, w1, w2, counts = pl.pallas_call(
        _router_kernel,
        out_shape=[
            jax.ShapeDtypeStruct((T, 1), jnp.int32),
            jax.ShapeDtypeStruct((T, 1), jnp.int32),
            jax.ShapeDtypeStruct((T, 128), jnp.float32),
            jax.ShapeDtypeStruct((T, 128), jnp.float32),
            jax.ShapeDtypeStruct((1, E), jnp.int32),
        ],
    )(scores, e_score_correction_bias.reshape(1, E))
    p1 = p1.reshape(T)
    p2 = p2.reshape(T)

    t_idx, e_idx, lo_r, hi_r, first_i = _step_table(counts.reshape(E))

    xb = x.astype(jnp.bfloat16)
    mesh = plsc.VectorSubcoreMesh(core_axis_name="c", subcore_axis_name="s")
    xs, ws = pl.kernel(
        _dispatch_body,
        out_type=[
            jax.ShapeDtypeStruct((R, H), jnp.float32),
            jax.ShapeDtypeStruct((R, 128), jnp.float32),
        ],
        mesh=mesh,
        scratch_types=[
            pltpu.VMEM((TPW,), jnp.int32),
            pltpu.VMEM((TPW,), jnp.int32),
            pltpu.VMEM((TPW, H), jnp.float32),
            pltpu.VMEM((TPW, 128), jnp.float32),
            pltpu.SemaphoreType.DMA,
        ],
    )(x, p1, p2, w1, w2)

    douts = pl.pallas_call(
        _gmm_kernel,
        grid_spec=pltpu.PrefetchScalarGridSpec(
            num_scalar_prefetch=5,
            grid=(S,),
            in_specs=[
                pl.BlockSpec((BT, H), lambda s, t, e, lo, hi, f: (t[s], 0)),
                pl.BlockSpec((BT, 128), lambda s, t, e, lo, hi, f: (t[s], 0)),
                pl.BlockSpec((1, FFN, H),
                             lambda s, t, e, lo, hi, f: (e[s], 0, 0)),
                pl.BlockSpec((1, FFN, H),
                             lambda s, t, e, lo, hi, f: (e[s], 0, 0)),
                pl.BlockSpec((1, H, FFN),
                             lambda s, t, e, lo, hi, f: (e[s], 0, 0)),
            ],
            out_specs=pl.BlockSpec((BT, H),
                                   lambda s, t, e, lo, hi, f: (t[s], 0)),
        ),
        out_shape=jax.ShapeDtypeStruct((R, H), jnp.float32),
    )(t_idx, e_idx, lo_r, hi_r, first_i, xs, ws, expert_gate, expert_up,
      expert_down)

    rp1, rp2 = pl.kernel(
        _combine_body,
        out_type=[
            jax.ShapeDtypeStruct((T, H), jnp.float32),
            jax.ShapeDtypeStruct((T, H), jnp.float32),
        ],
        mesh=mesh,
        scratch_types=[
            pltpu.VMEM((CH,), jnp.int32),
            pltpu.VMEM((CH,), jnp.int32),
            pltpu.VMEM((CH, H), jnp.float32),
            pltpu.VMEM((CH, H), jnp.float32),
            pltpu.SemaphoreType.DMA,
        ],
    )(douts, p1, p2)

    out = pl.pallas_call(
        _shared_kernel,
        grid=(NJS,),
        in_specs=[
            pl.BlockSpec((T, H), lambda j: (0, 0)),
            pl.BlockSpec((T, H), lambda j: (0, 0)),
            pl.BlockSpec((T, H), lambda j: (0, 0)),
            pl.BlockSpec((BC, H), lambda j: (j, 0)),
            pl.BlockSpec((BC, H), lambda j: (j, 0)),
            pl.BlockSpec((H, BC), lambda j: (0, j)),
        ],
        out_specs=pl.BlockSpec((T, H), lambda j: (0, 0)),
        out_shape=jax.ShapeDtypeStruct((T, H), jnp.float32),
    )(xb, rp1, rp2, shared_gate, shared_up, shared_down)

    return out.reshape(1, T, H)


# speedup vs baseline: 1.0814x; 1.0043x over previous
import functools

import jax
import jax.numpy as jnp
from jax import lax
from jax.experimental import pallas as pl
from jax.experimental.pallas import tpu as pltpu
from jax.experimental.pallas import tpu_sc as plsc

T = 2048
H = 1024
E = 8
FFN = 512
SFFN = 2048
NGROUP = 4
BC = 512
NJS = SFFN // BC
R = 2 * T
BT = 256
NT = R // BT
S = NT + E
NEG = -1e30

NC = 2
NS = 16
NW = NC * NS
TPW = T // NW
CH = 16
VPR = H // 16


def _router_kernel(scores_ref, bias_ref, p1_ref, p2_ref, w1_ref, w2_ref,
                   counts_ref):
    scores = scores_ref[...]
    sfc = scores + bias_ref[...]

    eidx = lax.broadcasted_iota(jnp.int32, (T, E), 1)
    gidx8 = eidx // 2
    gsum_full = jnp.zeros((T, E), jnp.float32)
    for g in range(NGROUP):
        in_g = gidx8 == g
        gsum_g = jnp.sum(jnp.where(in_g, sfc, 0.0), axis=-1, keepdims=True)
        gsum_full = jnp.where(in_g, gsum_g, gsum_full)

    m1 = jnp.max(gsum_full, axis=-1, keepdims=True)
    i1 = jnp.min(jnp.where(gsum_full == m1, gidx8, NGROUP),
                 axis=-1, keepdims=True)
    gs2 = jnp.where(gidx8 == i1, NEG, gsum_full)
    m2 = jnp.max(gs2, axis=-1, keepdims=True)
    i2 = jnp.min(jnp.where(gs2 == m2, gidx8, NGROUP),
                 axis=-1, keepdims=True)
    gsel = (gidx8 == i1) | (gidx8 == i2)
    masked = jnp.where(gsel, sfc, NEG)

    e_m1 = jnp.max(masked, axis=-1, keepdims=True)
    e1 = jnp.min(jnp.where(masked == e_m1, eidx, E), axis=-1, keepdims=True)
    masked2 = jnp.where(eidx == e1, NEG, masked)
    e_m2 = jnp.max(masked2, axis=-1, keepdims=True)
    e2 = jnp.min(jnp.where(masked2 == e_m2, eidx, E), axis=-1, keepdims=True)

    sel1 = (eidx == e1)
    sel2 = (eidx == e2)
    w1 = jnp.sum(jnp.where(sel1, scores, 0.0), axis=-1, keepdims=True)
    w2 = jnp.sum(jnp.where(sel2, scores, 0.0), axis=-1, keepdims=True)
    denom = w1 + w2 + 1e-20
    w1_ref[...] = jnp.broadcast_to(w1 / denom, (T, 128))
    w2_ref[...] = jnp.broadcast_to(w2 / denom, (T, 128))

    mask = (sel1 | sel2).astype(jnp.float32)
    r_t = lax.broadcasted_iota(jnp.int32, (T, T), 0)
    c_t = lax.broadcasted_iota(jnp.int32, (T, T), 1)
    ltri = (c_t < r_t).astype(jnp.float32)
    rank = lax.dot_general(ltri, mask, (((1,), (0,)), ((), ())),
                           preferred_element_type=jnp.float32)
    counts = jnp.sum(mask, axis=0, keepdims=True)
    lidx = lax.broadcasted_iota(jnp.int32, (1, E), 1)
    offs = jnp.zeros((1, E), jnp.float32)
    for e in range(1, E):
        off_e = jnp.sum(jnp.where(lidx < e, counts, 0.0),
                        axis=-1, keepdims=True)
        offs = jnp.where(lidx == e, off_e, offs)
    pos = offs + rank
    p1_ref[...] = jnp.sum(jnp.where(sel1, pos, 0.0), axis=-1,
                          keepdims=True).astype(jnp.int32)
    p2_ref[...] = jnp.sum(jnp.where(sel2, pos, 0.0), axis=-1,
                          keepdims=True).astype(jnp.int32)
    counts_ref[...] = counts.astype(jnp.int32)


def _gmm_kernel(tid_ref, eid_ref, lo_ref, hi_ref, first_ref,
                xs_ref, ws_ref, wg_ref, wu_ref, wd_ref, out_ref):
    s = pl.program_id(0)
    xs = xs_ref[...].astype(jnp.bfloat16)
    wg = wg_ref[0].astype(jnp.bfloat16)
    wu = wu_ref[0].astype(jnp.bfloat16)
    wd = wd_ref[0].astype(jnp.bfloat16)
    g = lax.dot_general(xs, wg, (((1,), (1,)), ((), ())),
                        preferred_element_type=jnp.float32)
    u = lax.dot_general(xs, wu, (((1,), (1,)), ((), ())),
                        preferred_element_type=jnp.float32)
    wv = jnp.max(ws_ref[...], axis=-1, keepdims=True)
    ridx = lax.broadcasted_iota(jnp.int32, (BT, FFN), 0)
    live = (ridx >= lo_ref[s]) & (ridx < hi_ref[s])
    a = jnp.where(live, g * jax.nn.sigmoid(g) * u * wv,
                  0.0).astype(jnp.bfloat16)
    partial = lax.dot_general(a, wd, (((1,), (1,)), ((), ())),
                              preferred_element_type=jnp.float32)

    @pl.when(first_ref[s] != 0)
    def _():
        out_ref[...] = partial

    @pl.when(first_ref[s] == 0)
    def _():
        out_ref[...] += partial


def _shared_kernel(x_ref, r1_ref, r2_ref, wg_ref, wu_ref, wd_ref, out_ref):
    x = x_ref[...]
    wg = wg_ref[...].astype(jnp.bfloat16)
    wu = wu_ref[...].astype(jnp.bfloat16)
    wd = wd_ref[...].astype(jnp.bfloat16)
    g = lax.dot_general(x, wg, (((1,), (1,)), ((), ())),
                        preferred_element_type=jnp.float32)
    u = lax.dot_general(x, wu, (((1,), (1,)), ((), ())),
                        preferred_element_type=jnp.float32)
    a = (g * jax.nn.sigmoid(g) * u).astype(jnp.bfloat16)
    partial = lax.dot_general(a, wd, (((1,), (1,)), ((), ())),
                              preferred_element_type=jnp.float32)

    @pl.when(pl.program_id(0) == 0)
    def _():
        out_ref[...] = partial + r1_ref[...] + r2_ref[...]

    @pl.when(pl.program_id(0) > 0)
    def _():
        out_ref[...] += partial


def _dispatch_body(x_hbm, p1_hbm, p2_hbm, w1_hbm, w2_hbm, xs_hbm, ws_hbm,
                   idx1_v, idx2_v, rows_v, w1rows_v, w2rows_v, sem):
    wid = lax.axis_index("s") * NC + lax.axis_index("c")
    base = wid * TPW
    pltpu.sync_copy(p1_hbm.at[pl.ds(base, TPW)], idx1_v)
    pltpu.sync_copy(p2_hbm.at[pl.ds(base, TPW)], idx2_v)
    pltpu.sync_copy(x_hbm.at[pl.ds(base, TPW)], rows_v)
    pltpu.sync_copy(w1_hbm.at[pl.ds(base, TPW)], w1rows_v)
    pltpu.sync_copy(w2_hbm.at[pl.ds(base, TPW)], w2rows_v)
    c1 = pltpu.async_copy(rows_v, xs_hbm.at[idx1_v], sem)
    c2 = pltpu.async_copy(rows_v, xs_hbm.at[idx2_v], sem)
    c3 = pltpu.async_copy(w1rows_v, ws_hbm.at[idx1_v], sem)
    c4 = pltpu.async_copy(w2rows_v, ws_hbm.at[idx2_v], sem)
    c1.wait()
    c2.wait()
    c3.wait()
    c4.wait()


def _combine_body(douts_hbm, p1_hbm, p2_hbm,
                  rp1_hbm, rp2_hbm, idx1_v, idx2_v, r1_v, r2_v, sem0, sem1):
    wid = lax.axis_index("s") * NC + lax.axis_index("c")
    nch = TPW // CH
    sems = (sem0, sem1)

    def fire(c, b):
        cbase = wid * TPW + c * CH
        pltpu.sync_copy(p1_hbm.at[pl.ds(cbase, CH)], idx1_v.at[b])
        pltpu.sync_copy(p2_hbm.at[pl.ds(cbase, CH)], idx2_v.at[b])
        return (pltpu.async_copy(douts_hbm.at[idx1_v.at[b]], r1_v.at[b],
                                 sems[b]),
                pltpu.async_copy(douts_hbm.at[idx2_v.at[b]], r2_v.at[b],
                                 sems[b]))

    pend = fire(0, 0)
    for c in range(nch):
        b = c % 2
        nxt = fire(c + 1, 1 - b) if c + 1 < nch else None
        pend[0].wait()
        pend[1].wait()
        cbase = wid * TPW + c * CH
        pltpu.sync_copy(r1_v.at[b], rp1_hbm.at[pl.ds(cbase, CH)])
        pltpu.sync_copy(r2_v.at[b], rp2_hbm.at[pl.ds(cbase, CH)])
        pend = nxt


def _step_table(counts):
    b = jnp.concatenate([jnp.zeros((1,), jnp.int32),
                         jnp.cumsum(counts)]).astype(jnp.int32)
    ii = jnp.arange(NT, dtype=jnp.int32)[:, None]
    ee = jnp.arange(E, dtype=jnp.int32)[None, :]
    lo = jnp.maximum(b[ee], BT * ii)
    hi = jnp.minimum(b[ee + 1], BT * ii + BT)
    act = lo < hi
    posn = jnp.cumsum(act.reshape(-1)).astype(jnp.int32) - 1
    wpos = jnp.where(act.reshape(-1), posn, S)
    first = act & (jnp.cumsum(act, axis=1) == 1)

    def scat(vals, fill):
        return jnp.full((S,), fill, jnp.int32).at[wpos].set(
            vals.reshape(-1).astype(jnp.int32), mode="drop")

    t_idx = scat(jnp.broadcast_to(ii, (NT, E)), NT - 1)
    e_idx = scat(jnp.broadcast_to(ee, (NT, E)), E - 1)
    lo_r = scat(lo - BT * ii, 0)
    hi_r = scat(hi - BT * ii, 0)
    first_i = scat(first, 0)
    return t_idx, e_idx, lo_r, hi_r, first_i


def kernel(hidden_states, gate_weight, e_score_correction_bias, expert_gate,
           expert_up, expert_down, shared_gate, shared_up, shared_down):
    x = hidden_states.reshape(T, H)
    scores = jax.nn.sigmoid(x @ gate_weight.astype(jnp.float32).T)

    p1, p2, w1, w2, counts = pl.pallas_call(
        _router_kernel,
        out_shape=[
            jax.ShapeDtypeStruct((T, 1), jnp.int32),
            jax.ShapeDtypeStruct((T, 1), jnp.int32),
            jax.ShapeDtypeStruct((T, 128), jnp.float32),
            jax.ShapeDtypeStruct((T, 128), jnp.float32),
            jax.ShapeDtypeStruct((1, E), jnp.int32),
        ],
    )(scores, e_score_correction_bias.reshape(1, E))
    p1 = p1.reshape(T)
    p2 = p2.reshape(T)

    t_idx, e_idx, lo_r, hi_r, first_i = _step_table(counts.reshape(E))

    xb = x.astype(jnp.bfloat16)
    mesh = plsc.VectorSubcoreMesh(core_axis_name="c", subcore_axis_name="s")
    xs, ws = pl.kernel(
        _dispatch_body,
        out_type=[
            jax.ShapeDtypeStruct((R, H), jnp.float32),
            jax.ShapeDtypeStruct((R, 128), jnp.float32),
        ],
        mesh=mesh,
        scratch_types=[
            pltpu.VMEM((TPW,), jnp.int32),
            pltpu.VMEM((TPW,), jnp.int32),
            pltpu.VMEM((TPW, H), jnp.float32),
            pltpu.VMEM((TPW, 128), jnp.float32),
            pltpu.VMEM((TPW, 128), jnp.float32),
            pltpu.SemaphoreType.DMA,
        ],
    )(x, p1, p2, w1, w2)

    douts = pl.pallas_call(
        _gmm_kernel,
        grid_spec=pltpu.PrefetchScalarGridSpec(
            num_scalar_prefetch=5,
            grid=(S,),
            in_specs=[
                pl.BlockSpec((BT, H), lambda s, t, e, lo, hi, f: (t[s], 0)),
                pl.BlockSpec((BT, 128), lambda s, t, e, lo, hi, f: (t[s], 0)),
                pl.BlockSpec((1, FFN, H),
                             lambda s, t, e, lo, hi, f: (e[s], 0, 0)),
                pl.BlockSpec((1, FFN, H),
                             lambda s, t, e, lo, hi, f: (e[s], 0, 0)),
                pl.BlockSpec((1, H, FFN),
                             lambda s, t, e, lo, hi, f: (e[s], 0, 0)),
            ],
            out_specs=pl.BlockSpec((BT, H),
                                   lambda s, t, e, lo, hi, f: (t[s], 0)),
        ),
        out_shape=jax.ShapeDtypeStruct((R, H), jnp.float32),
    )(t_idx, e_idx, lo_r, hi_r, first_i, xs, ws, expert_gate, expert_up,
      expert_down)

    rp1, rp2 = pl.kernel(
        _combine_body,
        out_type=[
            jax.ShapeDtypeStruct((T, H), jnp.float32),
            jax.ShapeDtypeStruct((T, H), jnp.float32),
        ],
        mesh=mesh,
        scratch_types=[
            pltpu.VMEM((2, CH), jnp.int32),
            pltpu.VMEM((2, CH), jnp.int32),
            pltpu.VMEM((2, CH, H), jnp.float32),
            pltpu.VMEM((2, CH, H), jnp.float32),
            pltpu.SemaphoreType.DMA,
            pltpu.SemaphoreType.DMA,
        ],
    )(douts, p1, p2)

    out = pl.pallas_call(
        _shared_kernel,
        grid=(NJS,),
        in_specs=[
            pl.BlockSpec((T, H), lambda j: (0, 0)),
            pl.BlockSpec((T, H), lambda j: (0, 0)),
            pl.BlockSpec((T, H), lambda j: (0, 0)),
            pl.BlockSpec((BC, H), lambda j: (j, 0)),
            pl.BlockSpec((BC, H), lambda j: (j, 0)),
            pl.BlockSpec((H, BC), lambda j: (0, j)),
        ],
        out_specs=pl.BlockSpec((T, H), lambda j: (0, 0)),
        out_shape=jax.ShapeDtypeStruct((T, H), jnp.float32),
    )(xb, rp1, rp2, shared_gate, shared_up, shared_down)

    return out.reshape(1, T, H)
